# Initial kernel scaffold; baseline (speedup 1.0000x reference)
import functools

import jax
import jax.numpy as jnp
from jax import lax
from jax.experimental import pallas as pl
from jax.experimental.pallas import tpu as pltpu
from jax.experimental.pallas import tpu_sc as plsc

VOCAB = 1000000
SIZE = 64
WINDOW = 5
NEG = 10
PAD = VOCAB
B = 16384

CW = 2 * WINDOW
DCOLS = 2 * CW + 2 + NEG
NC = 2
NS = 16
NW = NC * NS
L = 16
ROWS_W = B // NW
CHUNK = 32
NCHUNK = ROWS_W // CHUNK
NIDX = CHUNK * CW
NVEC = SIZE // L
PIECES = [(0, 128), (128, 128), (256, 64)]

_mesh = plsc.VectorSubcoreMesh(core_axis_name="c", subcore_axis_name="s")


@functools.partial(
    pl.kernel,
    mesh=_mesh,
    out_type=jax.ShapeDtypeStruct((NW, 2, L), jnp.float32),
    scratch_types=[
        pltpu.VMEM((ROWS_W, DCOLS), jnp.int32),
        pltpu.VMEM((2, NIDX), jnp.int32),
        pltpu.VMEM((2, NIDX), jnp.int32),
        pltpu.VMEM((2, CHUNK), jnp.int32),
        pltpu.VMEM((2, NIDX, SIZE), jnp.float32),
        pltpu.VMEM((2, NIDX, SIZE), jnp.float32),
        pltpu.VMEM((2, CHUNK, SIZE), jnp.float32),
        pltpu.VMEM((CHUNK,), jnp.float32),
        pltpu.VMEM((NIDX,), jnp.float32),
        pltpu.VMEM((2, L), jnp.float32),
        pltpu.SemaphoreType.DMA,
        pltpu.SemaphoreType.DMA,
    ],
)
def _cbow_sc(data_hbm, emb0_hbm, emb1_hbm, out_hbm,
             data_v, ctx_idx, neg_idx, word_idx,
             ctx_rows, neg_rows, word_rows,
             pos_ip, neg_ip, acc_v, sem0, sem1):
    wid = lax.axis_index("s") * NC + lax.axis_index("c")
    base = wid * ROWS_W
    sems = [sem0, sem1]
    iota = lax.iota(jnp.int32, L)

    pltpu.sync_copy(data_hbm.at[pl.ds(base, ROWS_W)], data_v)

    zero = jnp.zeros((L,), jnp.float32)
    acc_v[0, :] = zero
    acc_v[1, :] = zero

    def build_idx(g, buf):
        cbase = g * CHUNK
        for i in range(NIDX // L):
            p = i * L + iota
            r = cbase + p // CW
            c = p % CW
            ctx_idx[buf, pl.ds(i * L, L)] = plsc.load_gather(data_v, [r, c])
            neg_idx[buf, pl.ds(i * L, L)] = plsc.load_gather(
                data_v, [r, c + CW + 2])
        for i in range(CHUNK // L):
            r = cbase + i * L + iota
            c = jnp.full((L,), CW + 1, jnp.int32)
            word_idx[buf, pl.ds(i * L, L)] = plsc.load_gather(data_v, [r, c])

    def copies(buf, sem):
        out = []
        for off, n in PIECES:
            out.append(pltpu.make_async_copy(
                emb0_hbm.at[ctx_idx.at[buf, pl.ds(off, n)]],
                ctx_rows.at[buf, pl.ds(off, n)], sem))
            out.append(pltpu.make_async_copy(
                emb1_hbm.at[neg_idx.at[buf, pl.ds(off, n)]],
                neg_rows.at[buf, pl.ds(off, n)], sem))
        out.append(pltpu.make_async_copy(
            emb1_hbm.at[word_idx.at[buf]], word_rows.at[buf], sem))
        return out

    def fire(buf, sem):
        for c in copies(buf, sem):
            c.start()

    def drain(buf, sem):
        for c in copies(buf, sem):
            c.wait()

    def compute(g, buf):
        cbase = g * CHUNK

        def row_body(r, _):
            inv_len = 1.0 / data_v[cbase + r, CW].astype(jnp.float32)
            cm = []
            for k in range(NVEC):
                s = ctx_rows[buf, r * CW, pl.ds(k * L, L)]
                for w in range(1, CW):
                    s = s + ctx_rows[buf, r * CW + w, pl.ds(k * L, L)]
                cm.append(s * inv_len)
            dot = cm[0] * word_rows[buf, r, pl.ds(0, L)]
            for k in range(1, NVEC):
                dot = dot + cm[k] * word_rows[buf, r, pl.ds(k * L, L)]
            pos_ip[r] = jnp.sum(dot)
            for j in range(NEG):
                nd = cm[0] * neg_rows[buf, r * CW + j, pl.ds(0, L)]
                for k in range(1, NVEC):
                    nd = nd + cm[k] * neg_rows[buf, r * CW + j, pl.ds(k * L, L)]
                neg_ip[r * CW + j] = jnp.sum(nd)
            return 0

        lax.fori_loop(0, CHUNK, row_body, 0)

        accp = acc_v[0, :]
        for i in range(CHUNK // L):
            x = pos_ip[pl.ds(i * L, L)]
            t = 1.0 - 1.0 / (1.0 + jnp.exp(-x))
            accp = accp + t * t
        acc_v[0, :] = accp
        accn = acc_v[1, :]
        for i in range(NIDX // L):
            p = i * L + iota
            r = cbase + p // CW
            c = p % CW + CW + 2 + NEG
            m = plsc.load_gather(data_v, [r, c]).astype(jnp.float32)
            x = neg_ip[pl.ds(i * L, L)]
            v = m / (1.0 + jnp.exp(-x))
            accn = accn + v * v
        acc_v[1, :] = accn

    build_idx(0, 0)
    fire(0, sem0)

    def chunk_pair(g2, _):
        for par in range(2):
            g = g2 + par

            @pl.when(g + 1 < NCHUNK)
            def _():
                build_idx(g + 1, 1 - par)
                fire(1 - par, sems[1 - par])

            drain(par, sems[par])
            compute(g, par)
        return 0

    lax.fori_loop(0, NCHUNK // 2, lambda i, c: chunk_pair(i * 2, c), 0)

    pltpu.sync_copy(acc_v, out_hbm.at[wid])


def kernel(data, emb0, emb1):
    part = _cbow_sc(data.astype(jnp.int32), emb0, emb1)
    s = jnp.sum(part, axis=(0, 2)) * (0.5 / B)
    return (s[0], s[1])


# speedup vs baseline: 3.1736x; 3.1736x over previous
import functools

import jax
import jax.numpy as jnp
from jax import lax
from jax.experimental import pallas as pl
from jax.experimental.pallas import tpu as pltpu
from jax.experimental.pallas import tpu_sc as plsc

VOCAB = 1000000
SIZE = 64
WINDOW = 5
NEG = 10
PAD = VOCAB
B = 16384

CW = 2 * WINDOW
DCOLS = 2 * CW + 2 + NEG
NC = 2
NS = 16
NW = NC * NS
L = 16
ROWS_W = B // NW
CHUNK = 32
NCHUNK = ROWS_W // CHUNK
NIDX = CHUNK * CW
NVEC = SIZE // L
PIECES = [(0, 128), (128, 128), (256, 64)]

_mesh = plsc.VectorSubcoreMesh(core_axis_name="c", subcore_axis_name="s")


@functools.partial(
    pl.kernel,
    mesh=_mesh,
    compiler_params=pltpu.CompilerParams(
        use_tc_tiling_on_sc=False, needs_layout_passes=False),
    out_type=jax.ShapeDtypeStruct((NW, 2, L), jnp.float32),
    scratch_types=[
        pltpu.VMEM((ROWS_W * DCOLS,), jnp.int32),
        pltpu.VMEM((ROWS_W * CW,), jnp.int32),
        pltpu.VMEM((ROWS_W * CW,), jnp.int32),
        pltpu.VMEM((ROWS_W,), jnp.int32),
        pltpu.VMEM((2, NIDX, SIZE), jnp.float32),
        pltpu.VMEM((2, NIDX, SIZE), jnp.float32),
        pltpu.VMEM((2, CHUNK, SIZE), jnp.float32),
        pltpu.VMEM((2, L), jnp.float32),
        pltpu.SemaphoreType.DMA,
        pltpu.SemaphoreType.DMA,
    ],
)
def _cbow_sc(data_hbm, ctx_hbm, neg_hbm, word_hbm, emb0_hbm, emb1_hbm,
             out_hbm,
             data_v, ctx_idx_v, neg_idx_v, word_idx_v,
             ctx_rows, neg_rows, word_rows,
             acc_v, sem0, sem1):
    wid = lax.axis_index("s") * NC + lax.axis_index("c")
    base = wid * ROWS_W
    sems = [sem0, sem1]
    iota = lax.iota(jnp.int32, L)

    pltpu.sync_copy(data_hbm.at[pl.ds(base * DCOLS, ROWS_W * DCOLS)], data_v)
    pltpu.sync_copy(ctx_hbm.at[pl.ds(base * CW, ROWS_W * CW)], ctx_idx_v)
    pltpu.sync_copy(neg_hbm.at[pl.ds(base * CW, ROWS_W * CW)], neg_idx_v)
    pltpu.sync_copy(word_hbm.at[pl.ds(base, ROWS_W)], word_idx_v)

    zero = jnp.zeros((L,), jnp.float32)
    acc_v[0, :] = zero
    acc_v[1, :] = zero

    def copies(g, buf, sem):
        gbase = g * NIDX
        out = []
        for off, n in PIECES:
            out.append(pltpu.make_async_copy(
                emb0_hbm.at[ctx_idx_v.at[pl.ds(gbase + off, n)]],
                ctx_rows.at[buf, pl.ds(off, n)], sem))
            out.append(pltpu.make_async_copy(
                emb1_hbm.at[neg_idx_v.at[pl.ds(gbase + off, n)]],
                neg_rows.at[buf, pl.ds(off, n)], sem))
        out.append(pltpu.make_async_copy(
            emb1_hbm.at[word_idx_v.at[pl.ds(g * CHUNK, CHUNK)]],
            word_rows.at[buf], sem))
        return out

    def fire(g, buf, sem):
        for c in copies(g, buf, sem):
            c.start()

    def drain(g, buf, sem):
        for c in copies(g, buf, sem):
            c.wait()

    def compute(g, buf):
        cbase = g * CHUNK
        lane0 = iota == 0
        hi_lanes = iota >= (L - NEG)

        def row_body(r, carry):
            accp, accn = carry
            lvec = data_v[pl.ds((cbase + r) * DCOLS + 8, L)]
            len_s = jnp.sum(jnp.where(iota == 2, lvec, 0))
            inv_len = 1.0 / jnp.full((L,), len_s.astype(jnp.float32))
            mvec = data_v[pl.ds((cbase + r) * DCOLS + (DCOLS - L), L)]
            cm = []
            for k in range(NVEC):
                s = ctx_rows[buf, r * CW, pl.ds(k * L, L)]
                for w in range(1, CW):
                    s = s + ctx_rows[buf, r * CW + w, pl.ds(k * L, L)]
                cm.append(s * inv_len)
            dot = cm[0] * word_rows[buf, r, pl.ds(0, L)]
            for k in range(1, NVEC):
                dot = dot + cm[k] * word_rows[buf, r, pl.ds(k * L, L)]
            p = jnp.sum(dot)
            sig_p = 1.0 / (1.0 + jnp.exp(jnp.full((L,), -p)))
            t = jnp.where(lane0, 1.0 - sig_p, 0.0)
            accp = accp + t * t
            nv = jnp.zeros((L,), jnp.float32)
            for j in range(NEG):
                nd = cm[0] * neg_rows[buf, r * CW + j, pl.ds(0, L)]
                for k in range(1, NVEC):
                    nd = nd + cm[k] * neg_rows[buf, r * CW + j, pl.ds(k * L, L)]
                nj = jnp.sum(nd)
                nv = jnp.where(iota == (L - NEG + j), jnp.full((L,), nj), nv)
            m = jnp.where(hi_lanes, mvec, 0).astype(jnp.float32)
            v = m / (1.0 + jnp.exp(-nv))
            accn = accn + v * v
            return accp, accn

        accp, accn = lax.fori_loop(0, CHUNK, row_body,
                                   (acc_v[0, :], acc_v[1, :]))
        acc_v[0, :] = accp
        acc_v[1, :] = accn

    fire(0, 0, sem0)

    def chunk_pair(g2, _):
        for par in range(2):
            g = g2 + par

            @pl.when(g + 1 < NCHUNK)
            def _():
                fire(g + 1, 1 - par, sems[1 - par])

            drain(g, par, sems[par])
            compute(g, par)
        return 0

    lax.fori_loop(0, NCHUNK // 2, lambda i, c: chunk_pair(i * 2, c), 0)

    pltpu.sync_copy(acc_v, out_hbm.at[wid])


def kernel(data, emb0, emb1):
    data = data.astype(jnp.int32)
    ctx = data[:, 0:CW].reshape(-1)
    word = data[:, CW + 1]
    neg = data[:, CW + 2:CW + 2 + NEG].reshape(-1)
    part = _cbow_sc(data.reshape(-1), ctx, neg, word, emb0, emb1)
    s = jnp.sum(part, axis=(0, 2)) * (0.5 / B)
    return (s[0], s[1])
